# Initial kernel scaffold; baseline (speedup 1.0000x reference)
import jax
import jax.numpy as jnp
from jax.experimental import pallas as pl


def kernel(h, h_nn, W, b):
    raise NotImplementedError("write your pallas kernel here")


# speedup vs baseline: 1.1226x; 1.1226x over previous
import jax
import jax.numpy as jnp
from jax.experimental import pallas as pl

N = 10000
K = 32
D = 128
OUT = 128
BLOCK_M = 400


def _body(h_ref, hnn_ref, w1_ref, w2_ref, b_ref, o_ref):
    aggr = jnp.sum(hnn_ref[...], axis=1)
    o_ref[...] = (
        jnp.dot(h_ref[...], w1_ref[...], preferred_element_type=jnp.float32)
        + jnp.dot(aggr, w2_ref[...], preferred_element_type=jnp.float32)
        + b_ref[...]
    )


def kernel(h, h_nn, W, b):
    w1t = W[:, :D].T
    w2t = W[:, D:].T
    b2 = b.reshape(1, OUT)
    grid = (N // BLOCK_M,)
    return pl.pallas_call(
        _body,
        grid=grid,
        in_specs=[
            pl.BlockSpec((BLOCK_M, D), lambda i: (i, 0)),
            pl.BlockSpec((BLOCK_M, K, D), lambda i: (i, 0, 0)),
            pl.BlockSpec((D, OUT), lambda i: (0, 0)),
            pl.BlockSpec((D, OUT), lambda i: (0, 0)),
            pl.BlockSpec((1, OUT), lambda i: (0, 0)),
        ],
        out_specs=pl.BlockSpec((BLOCK_M, OUT), lambda i: (i, 0)),
        out_shape=jax.ShapeDtypeStruct((N, OUT), jnp.float32),
    )(h, h_nn, w1t, w2t, b2)
